# baseline (speedup 1.0000x reference)
import functools

import jax
import jax.numpy as jnp
from jax import lax
from jax.experimental import pallas as pl
from jax.experimental.pallas import tpu as pltpu
from jax.experimental.pallas import tpu_sc as plsc

D_MODEL = 2048
VOCAB = 100000
NUM_CLUSTERS = 1024
MAP_LEN = 256
N_PROBES = 64
K = N_PROBES * MAP_LEN

NC = 2
NS = 16
NW = NC * NS
LANES = 16
RPW = K // NW
G = 16
NCH = RPW // G
JU = 4


def _topk_kernel(h_ref, c_ref, top_out_ref):
    c = c_ref[...]
    sims = jnp.dot(h_ref[...], c, preferred_element_type=jnp.float32)
    norm2 = jnp.sum(c * c, axis=0, keepdims=True)
    sims = (sims * lax.rsqrt(norm2)).reshape(8, NUM_CLUSTERS // 8)
    idx8 = (lax.broadcasted_iota(jnp.int32, (8, 128), 0) * 128
            + lax.broadcasted_iota(jnp.int32, (8, 128), 1))

    def pick(va, ia, vb, ib):
        take = (vb > va) | ((vb == va) & (ib < ia))
        return jnp.where(take, vb, va), jnp.where(take, ib, ia)

    def body(i, s8):
        v, ids = s8, idx8
        for h in (4, 2, 1):
            v, ids = pick(v[:h, :], ids[:h, :], v[h:2 * h, :], ids[h:2 * h, :])
        for k in (1, 2, 4, 8, 16, 32, 64):
            v, ids = pick(v, ids, jnp.roll(v, k, axis=1),
                          jnp.roll(ids, k, axis=1))
        top_out_ref[pl.ds(i, 1), :] = ids
        return jnp.where(idx8 == ids, -jnp.inf, s8)

    lax.fori_loop(0, N_PROBES, body, sims)


def _argmax_kernel(l_ref, idx_ref, out_ref):
    l = l_ref[...]
    m = jnp.max(l)
    flat = (lax.broadcasted_iota(jnp.int32, (128, 128), 0) * 128
            + lax.broadcasted_iota(jnp.int32, (128, 128), 1))
    pos = jnp.min(jnp.where(l == m, flat, jnp.int32(1 << 30)))
    flat2 = (lax.broadcasted_iota(jnp.int32, (64, 256), 0) * 256
             + lax.broadcasted_iota(jnp.int32, (64, 256), 1))
    out_ref[0, 0] = jnp.max(jnp.where(flat2 == pos, idx_ref[...], -1))


def _lane_shuffle(t, idx):
    dnums = lax.GatherDimensionNumbers(
        offset_dims=(), collapsed_slice_dims=(0,), start_index_map=(0,))
    return lax.gather(t, idx[:, None], dnums, slice_sizes=(1,),
                      mode=lax.GatherScatterMode.PROMISE_IN_BOUNDS)


def _sc_body(w_hbm, maps_hbm, top_hbm, h_hbm, out_l_hbm, out_i_hbm,
             clbuf, idx2_v, maps_v, h_v, buf0, buf1, logit_v, sem0, sem1):
    wid = lax.axis_index("s") * NC + lax.axis_index("c")
    base = wid * RPW
    lane = lax.iota(jnp.int32, LANES)
    pltpu.sync_copy(h_hbm, h_v)

    pltpu.sync_copy(top_hbm.at[pl.ds(2 * wid, 2)], clbuf)
    cl0 = clbuf[0, pl.ds(0, LANES)][0]
    cl1 = clbuf[1, pl.ds(0, LANES)][0]
    idx2_v[...] = jnp.where(lane == 0, cl0, jnp.where(lane == 1, cl1, 0))
    pltpu.async_copy(maps_hbm.at[idx2_v.at[pl.ds(0, 2)]], maps_v,
                     sem0).wait()
    pltpu.sync_copy(maps_v, out_i_hbm.at[pl.ds(2 * wid, 2)])

    bufs = (buf0, buf1)
    sems = (sem0, sem1)

    def start(g, b):
        pltpu.async_copy(
            w_hbm.at[maps_v.at[g >> 4, pl.ds((g & 15) * G, G)]],
            bufs[b], sems[b])

    def wait(b):
        pltpu.make_async_copy(w_hbm.at[pl.ds(0, G)], bufs[b], sems[b]).wait()

    def compute(buf, g):
        zeros16 = tuple(jnp.zeros((LANES,), jnp.float32) for _ in range(G))

        @plsc.parallel_loop(0, D_MODEL // LANES, step=1, unroll=JU,
                            carry=zeros16)
        def jloop(j, accs):
            accs = list(accs)
            off = j * LANES
            hv = h_v[pl.ds(off, LANES)]
            for r in range(G):
                accs[r] = accs[r] + buf[r, pl.ds(off, LANES)] * hv
            return tuple(accs)

        lane = lax.iota(jnp.int32, LANES)
        res = jnp.zeros((LANES,), jnp.float32)
        for r in range(G):
            t = jloop[r]
            for s in (8, 4, 2, 1):
                t = t + _lane_shuffle(t, lane ^ s)
            res = jnp.where(lane == r, t, res)
        logit_v[pl.ds(g * G, G)] = res

    start(0, 0)

    def body(t, carry):
        g = t * 2
        start(g + 1, 1)
        wait(0)
        compute(buf0, g)

        @pl.when(t < NCH // 2 - 1)
        def _():
            start(g + 2, 0)

        wait(1)
        compute(buf1, g + 1)
        return carry

    lax.fori_loop(0, NCH // 2, body, 0)
    pltpu.sync_copy(logit_v, out_l_hbm.at[pl.ds(base, RPW)])


@functools.cache
def _sc_gather_dot():
    return pl.kernel(
        _sc_body,
        out_type=[jax.ShapeDtypeStruct((K,), jnp.float32),
                  jax.ShapeDtypeStruct((N_PROBES, MAP_LEN), jnp.int32)],
        mesh=plsc.VectorSubcoreMesh(core_axis_name="c", subcore_axis_name="s"),
        scratch_types=[
            pltpu.VMEM((2, 128), jnp.int32),
            pltpu.VMEM((LANES,), jnp.int32),
            pltpu.VMEM((2, MAP_LEN), jnp.int32),
            pltpu.VMEM((D_MODEL,), jnp.float32),
            pltpu.VMEM((G, D_MODEL), jnp.float32),
            pltpu.VMEM((G, D_MODEL), jnp.float32),
            pltpu.VMEM((RPW,), jnp.float32),
            pltpu.SemaphoreType.DMA,
            pltpu.SemaphoreType.DMA,
        ],
    )


def kernel(hidden_states, lm_head_weight, centroids, vocab_maps_tensor):
    h2d = hidden_states.reshape(1, D_MODEL)
    top = pl.pallas_call(
        _topk_kernel,
        out_shape=jax.ShapeDtypeStruct((N_PROBES, 128), jnp.int32),
    )(h2d, centroids)

    logits, idx = _sc_gather_dot()(lm_head_weight, vocab_maps_tensor, top,
                                   hidden_states.reshape(D_MODEL))

    out = pl.pallas_call(
        _argmax_kernel,
        out_shape=jax.ShapeDtypeStruct((1, 1), jnp.int32),
        out_specs=pl.BlockSpec(memory_space=pltpu.SMEM),
    )(logits.reshape(128, 128), idx)
    return out


# speedup vs baseline: 1.1105x; 1.1105x over previous
import functools

import jax
import jax.numpy as jnp
from jax import lax
from jax.experimental import pallas as pl
from jax.experimental.pallas import tpu as pltpu
from jax.experimental.pallas import tpu_sc as plsc

D_MODEL = 2048
VOCAB = 100000
NUM_CLUSTERS = 1024
MAP_LEN = 256
N_PROBES = 64
K = N_PROBES * MAP_LEN

NC = 2
NS = 16
NW = NC * NS
LANES = 16
RPW = K // NW
G = 16
NCH = RPW // G
JU = 4


def _topk_kernel(h_ref, c_ref, top_out_ref):
    c = c_ref[...]
    sims = jnp.dot(h_ref[...], c, preferred_element_type=jnp.float32)
    norm2 = jnp.sum(c * c, axis=0, keepdims=True)
    sims = sims * lax.rsqrt(norm2)
    iota = lax.broadcasted_iota(jnp.int32, (1, NUM_CLUSTERS), 1)

    def body(i, s):
        m = jnp.max(s)
        idx = jnp.min(jnp.where(s == m, iota, NUM_CLUSTERS))
        top_out_ref[pl.ds(i, 1), :] = jnp.full((1, 128), idx, jnp.int32)
        return jnp.where(iota == idx, -jnp.inf, s)

    lax.fori_loop(0, N_PROBES, body, sims)


def _argmax_kernel(l_ref, idx_ref, out_ref):
    l = l_ref[...]
    m = jnp.max(l)
    flat = (lax.broadcasted_iota(jnp.int32, (128, 128), 0) * 128
            + lax.broadcasted_iota(jnp.int32, (128, 128), 1))
    pos = jnp.min(jnp.where(l == m, flat, jnp.int32(1 << 30)))
    flat2 = (lax.broadcasted_iota(jnp.int32, (64, 256), 0) * 256
             + lax.broadcasted_iota(jnp.int32, (64, 256), 1))
    out_ref[0, 0] = jnp.max(jnp.where(flat2 == pos, idx_ref[...], -1))


def _lane_shuffle(t, idx):
    dnums = lax.GatherDimensionNumbers(
        offset_dims=(), collapsed_slice_dims=(0,), start_index_map=(0,))
    return lax.gather(t, idx[:, None], dnums, slice_sizes=(1,),
                      mode=lax.GatherScatterMode.PROMISE_IN_BOUNDS)


def _sc_body(w_hbm, maps_hbm, top_hbm, h_hbm, out_l_hbm, out_i_hbm,
             clbuf, idx2_v, maps_v, h_v, buf0, buf1, logit_v, sem0, sem1):
    wid = lax.axis_index("s") * NC + lax.axis_index("c")
    base = wid * RPW
    lane = lax.iota(jnp.int32, LANES)
    pltpu.sync_copy(h_hbm, h_v)

    pltpu.sync_copy(top_hbm.at[pl.ds(2 * wid, 2)], clbuf)
    cl0 = clbuf[0, pl.ds(0, LANES)][0]
    cl1 = clbuf[1, pl.ds(0, LANES)][0]
    idx2_v[...] = jnp.where(lane == 0, cl0, jnp.where(lane == 1, cl1, 0))
    pltpu.async_copy(maps_hbm.at[idx2_v.at[pl.ds(0, 2)]], maps_v,
                     sem0).wait()
    pltpu.sync_copy(maps_v, out_i_hbm.at[pl.ds(2 * wid, 2)])

    bufs = (buf0, buf1)
    sems = (sem0, sem1)

    def start(g, b):
        pltpu.async_copy(
            w_hbm.at[maps_v.at[g >> 4, pl.ds((g & 15) * G, G)]],
            bufs[b], sems[b])

    def wait(b):
        pltpu.make_async_copy(w_hbm.at[pl.ds(0, G)], bufs[b], sems[b]).wait()

    def compute(buf, g):
        zeros16 = tuple(jnp.zeros((LANES,), jnp.float32) for _ in range(G))

        @plsc.parallel_loop(0, D_MODEL // LANES, step=1, unroll=JU,
                            carry=zeros16)
        def jloop(j, accs):
            accs = list(accs)
            off = j * LANES
            hv = h_v[pl.ds(off, LANES)]
            for r in range(G):
                accs[r] = accs[r] + buf[r, pl.ds(off, LANES)] * hv
            return tuple(accs)

        lane = lax.iota(jnp.int32, LANES)
        res = jnp.zeros((LANES,), jnp.float32)
        for r in range(G):
            t = jloop[r]
            for s in (8, 4, 2, 1):
                t = t + _lane_shuffle(t, lane ^ s)
            res = jnp.where(lane == r, t, res)
        logit_v[pl.ds(g * G, G)] = res

    start(0, 0)

    def body(t, carry):
        g = t * 2
        start(g + 1, 1)
        wait(0)
        compute(buf0, g)

        @pl.when(t < NCH // 2 - 1)
        def _():
            start(g + 2, 0)

        wait(1)
        compute(buf1, g + 1)
        return carry

    lax.fori_loop(0, NCH // 2, body, 0)
    pltpu.sync_copy(logit_v, out_l_hbm.at[pl.ds(base, RPW)])


@functools.cache
def _sc_gather_dot():
    return pl.kernel(
        _sc_body,
        out_type=[jax.ShapeDtypeStruct((K,), jnp.float32),
                  jax.ShapeDtypeStruct((N_PROBES, MAP_LEN), jnp.int32)],
        mesh=plsc.VectorSubcoreMesh(core_axis_name="c", subcore_axis_name="s"),
        scratch_types=[
            pltpu.VMEM((2, 128), jnp.int32),
            pltpu.VMEM((LANES,), jnp.int32),
            pltpu.VMEM((2, MAP_LEN), jnp.int32),
            pltpu.VMEM((D_MODEL,), jnp.float32),
            pltpu.VMEM((G, D_MODEL), jnp.float32),
            pltpu.VMEM((G, D_MODEL), jnp.float32),
            pltpu.VMEM((RPW,), jnp.float32),
            pltpu.SemaphoreType.DMA,
            pltpu.SemaphoreType.DMA,
        ],
    )


def kernel(hidden_states, lm_head_weight, centroids, vocab_maps_tensor):
    h2d = hidden_states.reshape(1, D_MODEL)
    top = pl.pallas_call(
        _topk_kernel,
        out_shape=jax.ShapeDtypeStruct((N_PROBES, 128), jnp.int32),
    )(h2d, centroids)

    logits, idx = _sc_gather_dot()(lm_head_weight, vocab_maps_tensor, top,
                                   hidden_states.reshape(D_MODEL))

    out = pl.pallas_call(
        _argmax_kernel,
        out_shape=jax.ShapeDtypeStruct((1, 1), jnp.int32),
        out_specs=pl.BlockSpec(memory_space=pltpu.SMEM),
    )(logits.reshape(128, 128), idx)
    return out


# speedup vs baseline: 1.2946x; 1.1657x over previous
import functools

import jax
import jax.numpy as jnp
from jax import lax
from jax.experimental import pallas as pl
from jax.experimental.pallas import tpu as pltpu
from jax.experimental.pallas import tpu_sc as plsc

D_MODEL = 2048
VOCAB = 100000
NUM_CLUSTERS = 1024
MAP_LEN = 256
N_PROBES = 64
K = N_PROBES * MAP_LEN

NC = 2
NS = 16
NW = NC * NS
LANES = 16
RPW = K // NW
G = 16
NCH = RPW // G
JU = 4


def _topk_kernel(h_ref, c_ref, mask_out_ref):
    c = c_ref[...]
    sims = jnp.dot(h_ref[...], c, preferred_element_type=jnp.float32)
    norm2 = jnp.sum(c * c, axis=0, keepdims=True)
    s = sims * lax.rsqrt(norm2)
    bits = lax.bitcast_convert_type(s, jnp.int32)
    mono = jnp.where(bits >= 0, bits, bits ^ jnp.int32(0x7FFFFFFF))

    def body(i, lohi):
        lo, hi = lohi
        mid = (lo >> 1) + (hi >> 1) + (lo & hi & 1)
        ge = jnp.sum(jnp.where(mono >= mid, 1, 0)) >= N_PROBES
        return jnp.where(ge, mid, lo), jnp.where(ge, hi, mid)

    lo, _ = lax.fori_loop(
        0, 32, body, (jnp.int32(-(2 ** 31)), jnp.int32(2 ** 31 - 1)))
    mask_out_ref[...] = jnp.where(mono >= lo, 1, 0).astype(jnp.int32)


def _argmax_kernel(l_ref, idx_ref, out_ref):
    l = l_ref[...]
    m = jnp.max(l)
    flat = (lax.broadcasted_iota(jnp.int32, (128, 128), 0) * 128
            + lax.broadcasted_iota(jnp.int32, (128, 128), 1))
    pos = jnp.min(jnp.where(l == m, flat, jnp.int32(1 << 30)))
    flat2 = (lax.broadcasted_iota(jnp.int32, (64, 256), 0) * 256
             + lax.broadcasted_iota(jnp.int32, (64, 256), 1))
    out_ref[0, 0] = jnp.max(jnp.where(flat2 == pos, idx_ref[...], -1))


def _lane_shuffle(t, idx):
    dnums = lax.GatherDimensionNumbers(
        offset_dims=(), collapsed_slice_dims=(0,), start_index_map=(0,))
    return lax.gather(t, idx[:, None], dnums, slice_sizes=(1,),
                      mode=lax.GatherScatterMode.PROMISE_IN_BOUNDS)


def _sc_body(w_hbm, maps_hbm, mask_hbm, h_hbm, out_l_hbm, out_i_hbm,
             mask_v, idx2_v, maps_v, h_v, buf0, buf1, logit_v, sem0, sem1):
    wid = lax.axis_index("s") * NC + lax.axis_index("c")
    base = wid * RPW
    lane = lax.iota(jnp.int32, LANES)
    pltpu.sync_copy(h_hbm, h_v)

    pltpu.sync_copy(mask_hbm, mask_v)
    ra = 2 * wid + 1
    rb = ra + 1
    carry = jnp.int32(0)
    ida = jnp.zeros((LANES,), jnp.int32)
    idb = jnp.zeros((LANES,), jnp.int32)
    for blk in range(NUM_CLUSTERS // LANES):
        v = mask_v[pl.ds(blk * LANES, LANES)]
        p = v
        for s in (1, 2, 4, 8):
            sh = _lane_shuffle(p, jnp.maximum(lane - s, 0))
            p = p + jnp.where(lane >= s, sh, 0)
        p = p + carry
        gid = blk * LANES + lane
        ida = ida + jnp.where((p == ra) & (v > 0), gid, 0)
        idb = idb + jnp.where((p == rb) & (v > 0), gid, 0)
        carry = p[LANES - 1]
    for s in (8, 4, 2, 1):
        ida = ida + _lane_shuffle(ida, lane ^ s)
        idb = idb + _lane_shuffle(idb, lane ^ s)
    idx2_v[...] = jnp.where(lane == 0, ida, jnp.where(lane == 1, idb, 0))
    pltpu.async_copy(maps_hbm.at[idx2_v.at[pl.ds(0, 2)]], maps_v,
                     sem0).wait()
    pltpu.sync_copy(maps_v, out_i_hbm.at[pl.ds(2 * wid, 2)])

    bufs = (buf0, buf1)
    sems = (sem0, sem1)

    def start(g, b):
        pltpu.async_copy(
            w_hbm.at[maps_v.at[g >> 4, pl.ds((g & 15) * G, G)]],
            bufs[b], sems[b])

    def wait(b):
        pltpu.make_async_copy(w_hbm.at[pl.ds(0, G)], bufs[b], sems[b]).wait()

    def compute(buf, g):
        zeros16 = tuple(jnp.zeros((LANES,), jnp.float32) for _ in range(G))

        @plsc.parallel_loop(0, D_MODEL // LANES, step=1, unroll=JU,
                            carry=zeros16)
        def jloop(j, accs):
            accs = list(accs)
            off = j * LANES
            hv = h_v[pl.ds(off, LANES)]
            for r in range(G):
                accs[r] = accs[r] + buf[r, pl.ds(off, LANES)] * hv
            return tuple(accs)

        lane = lax.iota(jnp.int32, LANES)
        res = jnp.zeros((LANES,), jnp.float32)
        for r in range(G):
            t = jloop[r]
            for s in (8, 4, 2, 1):
                t = t + _lane_shuffle(t, lane ^ s)
            res = jnp.where(lane == r, t, res)
        logit_v[pl.ds(g * G, G)] = res

    start(0, 0)

    def body(t, carry):
        g = t * 2
        start(g + 1, 1)
        wait(0)
        compute(buf0, g)

        @pl.when(t < NCH // 2 - 1)
        def _():
            start(g + 2, 0)

        wait(1)
        compute(buf1, g + 1)
        return carry

    lax.fori_loop(0, NCH // 2, body, 0)
    pltpu.sync_copy(logit_v, out_l_hbm.at[pl.ds(base, RPW)])


@functools.cache
def _sc_gather_dot():
    return pl.kernel(
        _sc_body,
        out_type=[jax.ShapeDtypeStruct((K,), jnp.float32),
                  jax.ShapeDtypeStruct((N_PROBES, MAP_LEN), jnp.int32)],
        mesh=plsc.VectorSubcoreMesh(core_axis_name="c", subcore_axis_name="s"),
        scratch_types=[
            pltpu.VMEM((NUM_CLUSTERS,), jnp.int32),
            pltpu.VMEM((LANES,), jnp.int32),
            pltpu.VMEM((2, MAP_LEN), jnp.int32),
            pltpu.VMEM((D_MODEL,), jnp.float32),
            pltpu.VMEM((G, D_MODEL), jnp.float32),
            pltpu.VMEM((G, D_MODEL), jnp.float32),
            pltpu.VMEM((RPW,), jnp.float32),
            pltpu.SemaphoreType.DMA,
            pltpu.SemaphoreType.DMA,
        ],
    )


def kernel(hidden_states, lm_head_weight, centroids, vocab_maps_tensor):
    h2d = hidden_states.reshape(1, D_MODEL)
    mask = pl.pallas_call(
        _topk_kernel,
        out_shape=jax.ShapeDtypeStruct((1, NUM_CLUSTERS), jnp.int32),
    )(h2d, centroids)

    logits, idx = _sc_gather_dot()(lm_head_weight, vocab_maps_tensor,
                                   mask.reshape(NUM_CLUSTERS),
                                   hidden_states.reshape(D_MODEL))

    out = pl.pallas_call(
        _argmax_kernel,
        out_shape=jax.ShapeDtypeStruct((1, 1), jnp.int32),
        out_specs=pl.BlockSpec(memory_space=pltpu.SMEM),
    )(logits.reshape(128, 128), idx)
    return out


# speedup vs baseline: 1.2968x; 1.0017x over previous
import functools

import jax
import jax.numpy as jnp
from jax import lax
from jax.experimental import pallas as pl
from jax.experimental.pallas import tpu as pltpu
from jax.experimental.pallas import tpu_sc as plsc

D_MODEL = 2048
VOCAB = 100000
NUM_CLUSTERS = 1024
MAP_LEN = 256
N_PROBES = 64
K = N_PROBES * MAP_LEN

NC = 2
NS = 16
NW = NC * NS
LANES = 16
RPW = K // NW
G = 16
NCH = RPW // G
JU = 4


def _topk_kernel(h_ref, c_ref, mask_out_ref):
    c = c_ref[...]
    sims = jnp.dot(h_ref[...], c, preferred_element_type=jnp.float32)
    norm2 = jnp.sum(c * c, axis=0, keepdims=True)
    s = sims * lax.rsqrt(norm2)
    bits = lax.bitcast_convert_type(s, jnp.int32)
    mono = jnp.where(bits >= 0, bits, bits ^ jnp.int32(0x7FFFFFFF))

    def body(i, lohi):
        lo, hi = lohi
        mid = (lo >> 1) + (hi >> 1) + (lo & hi & 1)
        ge = jnp.sum(jnp.where(mono >= mid, 1, 0)) >= N_PROBES
        return jnp.where(ge, mid, lo), jnp.where(ge, hi, mid)

    lo, _ = lax.fori_loop(
        0, 32, body, (jnp.int32(-(2 ** 31)), jnp.int32(2 ** 31 - 1)))
    mask_out_ref[...] = jnp.where(mono >= lo, 1, 0).astype(jnp.int32)


def _argmax_kernel(l_ref, idx_ref, out_ref):
    l = l_ref[...]
    m = jnp.max(l)
    flat = (lax.broadcasted_iota(jnp.int32, (128, 128), 0) * 128
            + lax.broadcasted_iota(jnp.int32, (128, 128), 1))
    pos = jnp.min(jnp.where(l == m, flat, jnp.int32(1 << 30)))
    flat2 = (lax.broadcasted_iota(jnp.int32, (64, 256), 0) * 256
             + lax.broadcasted_iota(jnp.int32, (64, 256), 1))
    out_ref[0, 0] = jnp.max(jnp.where(flat2 == pos, idx_ref[...], -1))


def _lane_shuffle(t, idx):
    dnums = lax.GatherDimensionNumbers(
        offset_dims=(), collapsed_slice_dims=(0,), start_index_map=(0,))
    return lax.gather(t, idx[:, None], dnums, slice_sizes=(1,),
                      mode=lax.GatherScatterMode.PROMISE_IN_BOUNDS)


def _sc_body(w_hbm, maps_hbm, mask_hbm, h_hbm, out_l_hbm, out_i_hbm,
             mask_v, idx2_v, maps_v, h_v, buf0, buf1, logit_v, sem0, sem1):
    wid = lax.axis_index("s") * NC + lax.axis_index("c")
    base = wid * RPW
    lane = lax.iota(jnp.int32, LANES)
    pltpu.sync_copy(h_hbm, h_v)

    pltpu.sync_copy(mask_hbm, mask_v)
    ra = 2 * wid + 1
    rb = ra + 1
    carry = jnp.int32(0)
    ida = jnp.zeros((LANES,), jnp.int32)
    idb = jnp.zeros((LANES,), jnp.int32)
    for blk in range(NUM_CLUSTERS // LANES):
        v = mask_v[0, pl.ds(blk * LANES, LANES)]
        p = v
        for s in (1, 2, 4, 8):
            sh = _lane_shuffle(p, jnp.maximum(lane - s, 0))
            p = p + jnp.where(lane >= s, sh, 0)
        p = p + carry
        gid = blk * LANES + lane
        ida = ida + jnp.where((p == ra) & (v > 0), gid, 0)
        idb = idb + jnp.where((p == rb) & (v > 0), gid, 0)
        carry = p[LANES - 1]
    for s in (8, 4, 2, 1):
        ida = ida + _lane_shuffle(ida, lane ^ s)
        idb = idb + _lane_shuffle(idb, lane ^ s)
    idx2_v[...] = jnp.where(lane == 0, ida, jnp.where(lane == 1, idb, 0))
    pltpu.async_copy(maps_hbm.at[idx2_v.at[pl.ds(0, 2)]], maps_v,
                     sem0).wait()
    pltpu.sync_copy(maps_v, out_i_hbm.at[pl.ds(2 * wid, 2)])

    bufs = (buf0, buf1)
    sems = (sem0, sem1)

    def start(g, b):
        pltpu.async_copy(
            w_hbm.at[maps_v.at[g >> 4, pl.ds((g & 15) * G, G)]],
            bufs[b], sems[b])

    def wait(b):
        pltpu.make_async_copy(w_hbm.at[pl.ds(0, G)], bufs[b], sems[b]).wait()

    def compute(buf, g):
        zeros16 = tuple(jnp.zeros((LANES,), jnp.float32) for _ in range(G))

        @plsc.parallel_loop(0, D_MODEL // LANES, step=1, unroll=JU,
                            carry=zeros16)
        def jloop(j, accs):
            accs = list(accs)
            off = j * LANES
            hv = h_v[pl.ds(off, LANES)]
            for r in range(G):
                accs[r] = accs[r] + buf[r, pl.ds(off, LANES)] * hv
            return tuple(accs)

        lane = lax.iota(jnp.int32, LANES)
        res = jnp.zeros((LANES,), jnp.float32)
        for r in range(G):
            t = jloop[r]
            for s in (8, 4, 2, 1):
                t = t + _lane_shuffle(t, lane ^ s)
            res = jnp.where(lane == r, t, res)
        logit_v[g >> 3, pl.ds((g & 7) * G, G)] = res

    start(0, 0)

    def body(t, carry):
        g = t * 2
        start(g + 1, 1)
        wait(0)
        compute(buf0, g)

        @pl.when(t < NCH // 2 - 1)
        def _():
            start(g + 2, 0)

        wait(1)
        compute(buf1, g + 1)
        return carry

    lax.fori_loop(0, NCH // 2, body, 0)
    pltpu.sync_copy(logit_v, out_l_hbm.at[pl.ds(4 * wid, 4)])


@functools.cache
def _sc_gather_dot():
    return pl.kernel(
        _sc_body,
        out_type=[jax.ShapeDtypeStruct((128, 128), jnp.float32),
                  jax.ShapeDtypeStruct((N_PROBES, MAP_LEN), jnp.int32)],
        mesh=plsc.VectorSubcoreMesh(core_axis_name="c", subcore_axis_name="s"),
        scratch_types=[
            pltpu.VMEM((1, NUM_CLUSTERS), jnp.int32),
            pltpu.VMEM((LANES,), jnp.int32),
            pltpu.VMEM((2, MAP_LEN), jnp.int32),
            pltpu.VMEM((D_MODEL,), jnp.float32),
            pltpu.VMEM((G, D_MODEL), jnp.float32),
            pltpu.VMEM((G, D_MODEL), jnp.float32),
            pltpu.VMEM((4, 128), jnp.float32),
            pltpu.SemaphoreType.DMA,
            pltpu.SemaphoreType.DMA,
        ],
    )


def kernel(hidden_states, lm_head_weight, centroids, vocab_maps_tensor):
    h2d = hidden_states.reshape(1, D_MODEL)
    mask = pl.pallas_call(
        _topk_kernel,
        out_shape=jax.ShapeDtypeStruct((1, NUM_CLUSTERS), jnp.int32),
    )(h2d, centroids)

    logits, idx = _sc_gather_dot()(lm_head_weight, vocab_maps_tensor,
                                   mask, hidden_states.reshape(D_MODEL))

    out = pl.pallas_call(
        _argmax_kernel,
        out_shape=jax.ShapeDtypeStruct((1, 1), jnp.int32),
        out_specs=pl.BlockSpec(memory_space=pltpu.SMEM),
    )(logits, idx)
    return out
